# baseline (speedup 1.0000x reference)
import functools
import numpy as np
import jax
import jax.numpy as jnp
from jax import lax
from jax.experimental import pallas as pl
from jax.experimental.pallas import tpu as pltpu
from jax.experimental.pallas import tpu_sc as plsc

N = 10000
E_NET = 80000
E_CELL = 80000
IN_NF = 128
OUT_NF = 8
NL = 4
L = 7
DUP = 4
H1 = 32
H2 = 32
N_PI = 1000
NU = NL * DUP
AXIS_W = 1 + 2 * L
L2 = L * L
AXIS_LEN = NL * AXIS_W

B_NET = 2000
B_CELL = 640
B_NODE = 2000

_E1 = np.zeros((L, L2), np.float32)
_E2 = np.zeros((L, L2), np.float32)
for _i in range(L):
    for _j in range(L):
        _E1[_i, _i * L + _j] = 1.0
        _E2[_j, _i * L + _j] = 1.0


def _leaky(x):
    return jnp.maximum(x, 0.2 * x)


def _dot(x, w):
    return jax.lax.dot_general(x, w, (((1,), (0,)), ((), ())),
                               preferred_element_type=jnp.float32)


def _mm(x, w, b):
    return _dot(x, w) + b


def _full_spec(shape):
    return pl.BlockSpec(shape, lambda i: tuple(0 for _ in shape))


def _wspecs(params):
    specs = []
    for (w, b) in params:
        specs.append(_full_spec(w.shape))
        specs.append(_full_spec((1, b.shape[0])))
    return specs


def _wargs(params):
    args = []
    for (w, b) in params:
        args.append(w)
        args.append(b.reshape(1, -1))
    return args


GK = 128
NCH = E_NET // GK
NWORK = 32
MAXJ = (NCH + NWORK - 1) // NWORK


def _sc_gather_body(netT_h, srcT_h, dstT_h,
                    ns_h, nd_h, cs_h, cd_h,
                    oNS_h, oND_h, oCS_h, oCD_h,
                    idx_v, b64_a, b64_b, b128_a, b128_b,
                    sem_a, sem_b):
    wid = lax.axis_index("s") * 2 + lax.axis_index("c")
    streams = [
        (netT_h, ns_h, oNS_h, (b64_a, b64_b)),
        (netT_h, nd_h, oND_h, (b64_a, b64_b)),
        (srcT_h, cs_h, oCS_h, (b128_a, b128_b)),
        (dstT_h, cd_h, oCD_h, (b128_a, b128_b)),
    ]
    semb = (sem_a, sem_b)
    for tab, idx2, out, bufs in streams:
        for j in range(MAXJ):
            ch = j * NWORK + wid

            @pl.when(ch < NCH)
            def _(ch=ch, tab=tab, idx2=idx2, out=out,
                  buf=bufs[j % 2], sem=semb[j % 2]):
                pltpu.sync_copy(idx2.at[ch], idx_v)
                pltpu.async_copy(tab.at[idx_v], buf, sem).wait()
                pltpu.sync_copy(buf, out.at[pl.ds(ch * GK, GK)])


def _sc_gather(netT, srcT, dstT, net_src, net_dst, cell_src, cell_dst):
    f32 = jnp.float32
    i32 = jnp.int32
    idx2 = [a.reshape(NCH, GK).astype(i32)
            for a in (net_src, net_dst, cell_src, cell_dst)]
    mesh = plsc.VectorSubcoreMesh(core_axis_name="c", subcore_axis_name="s")
    k = functools.partial(
        pl.kernel, mesh=mesh,
        out_type=[jax.ShapeDtypeStruct((E_NET, 128), f32),
                  jax.ShapeDtypeStruct((E_NET, 128), f32),
                  jax.ShapeDtypeStruct((E_CELL, 128), f32),
                  jax.ShapeDtypeStruct((E_CELL, 128), f32)],
        scratch_types=[pltpu.VMEM((GK,), i32),
                       pltpu.VMEM((GK, 128), f32),
                       pltpu.VMEM((GK, 128), f32),
                       pltpu.VMEM((GK, 128), f32),
                       pltpu.VMEM((GK, 128), f32),
                       pltpu.SemaphoreType.DMA,
                       pltpu.SemaphoreType.DMA],
    )(_sc_gather_body)
    return k(netT, srcT, dstT, *idx2)


WN = 16
WC = 48
ZCH = 10


def _sc_scat_body(vals_h, dst_h, z_h, out_h, idx_v, vb, acc):
    cid = lax.axis_index("c")
    sid = lax.axis_index("s")
    wid = sid * 2 + cid

    @pl.when(sid < ZCH)
    def _():
        pltpu.sync_copy(z_h.at[pl.ds(sid * 1000, 1000)],
                        acc.at[pl.ds(sid * 1000, 1000)])

    plsc.subcore_barrier()

    for j in range(MAXJ):
        ch = j * NWORK + wid

        @pl.when(ch < NCH)
        def _(ch=ch):
            pltpu.sync_copy(vals_h.at[pl.ds(ch * GK, GK)], vb)
            pltpu.sync_copy(dst_h.at[ch], idx_v)
            pltpu.sync_copy(vb, acc.at[idx_v], add=True)

    plsc.subcore_barrier()

    @pl.when(sid < ZCH)
    def _():
        pltpu.sync_copy(acc.at[pl.ds(sid * 1000, 1000)],
                        out_h.at[cid, pl.ds(sid * 1000, 1000)])


def _sc_scatter(vals128, dst):
    f32 = jnp.float32
    dst2 = dst.reshape(NCH, GK).astype(jnp.int32)
    mesh = plsc.VectorSubcoreMesh(core_axis_name="c", subcore_axis_name="s")
    k = functools.partial(
        pl.kernel, mesh=mesh,
        out_type=jax.ShapeDtypeStruct((2, N, 128), f32),
        scratch_types=[pltpu.VMEM((GK,), jnp.int32),
                       pltpu.VMEM((GK, 128), f32),
                       pltpu.VMEM_SHARED((N, 128), f32)],
    )(_sc_scat_body)
    return k(vals128, dst2, jnp.zeros((N, 128), f32))


def _proj_body(nf_ref, ats_ref, nw_ref, qw_ref, cw_ref,
               netT_ref, srcT_ref, dstT_ref):
    nf = nf_ref[...]
    ats = ats_ref[...]
    nw = nw_ref[...]
    qw = qw_ref[...]
    cw = cw_ref[...]
    netS = _dot(ats, nw[0:OUT_NF, :]) + _dot(nf, nw[OUT_NF:OUT_NF + IN_NF, :])
    netD = _dot(nf, nw[OUT_NF + IN_NF:, :])
    netT_ref[...] = jnp.concatenate([netS, netD], axis=1)
    qS = _dot(ats, qw[0:OUT_NF, :]) + _dot(nf, qw[OUT_NF:OUT_NF + IN_NF, :])
    qD = _dot(nf, qw[OUT_NF + IN_NF:, :])
    cS = _dot(ats, cw[0:OUT_NF, :]) + _dot(nf, cw[OUT_NF:OUT_NF + IN_NF, :])
    cD = _dot(nf, cw[OUT_NF + IN_NF:, :])
    srcT_ref[...] = jnp.concatenate([qS, cS], axis=1)
    dstT_ref[...] = jnp.concatenate([qD, cD], axis=1)


def _net_body(preA_ref, preB_ref, b0, w1, b1, w2, b2, w3, b3, w4, b4, out_ref):
    h = _leaky(preA_ref[:, 0:64] + preB_ref[:, 64:128] + b0[...])
    h = _leaky(_mm(h, w1[...], b1[...]))
    h = _leaky(_mm(h, w2[...], b2[...]))
    h = _leaky(_mm(h, w3[...], b3[...]))
    efn = _mm(h, w4[...], b4[...])
    B = efn.shape[0]
    out_ref[...] = jnp.concatenate(
        [efn, jnp.ones((B, 1), jnp.float32),
         jnp.zeros((B, 128 - OUT_NF - 1), jnp.float32)], axis=1)


def _cell_body(preA_ref, preB_ref, ef_ref, e1_ref, e2_ref, o49_ref,
               qb0, q1, qb1, q2, qb2, q3, qb3,
               a0, a1, ab1, a2, ab2, a3, ab3,
               cwr, cb0, c1, cb1, c2, cb2, c3, cb3,
               efc1_ref, efc2_ref, x1_scr):
    pre = preA_ref[...] + preB_ref[...]
    ef = ef_ref[...]

    h = _leaky(pre[:, 0:64] + qb0[...])
    h = _leaky(_mm(h, q1[...], qb1[...]))
    h = _leaky(_mm(h, q2[...], qb2[...]))
    q = _mm(h, q3[...], qb3[...])

    B = q.shape[0]

    @pl.when(pl.program_id(0) == 0)
    def _():
        x1_scr[:, 2 + AXIS_W:] = jnp.ones((NU * B, 1), jnp.float32)

    for p in range(NU):
        nl = p // DUP
        x1_scr[p * B:(p + 1) * B, 0:2] = q[:, 2 * p:2 * p + 2]
        x1_scr[p * B:(p + 1) * B, 2:2 + AXIS_W] = (
            ef[:, nl * AXIS_W:(nl + 1) * AXIS_W])

    xa = x1_scr[...]
    h = _leaky(_dot(xa, a0[...]))
    h = _leaky(_mm(h, a1[...], ab1[...]))
    h = _leaky(_mm(h, a2[...], ab2[...]))
    a4 = _mm(h, a3[...], ab3[...])

    ax = a4[:, 0:L]
    ay = a4[:, L:2 * L]
    outer = _dot(ax, e1_ref[...]) * _dot(ay, e2_ref[...])
    o49 = o49_ref[...]
    rcols = []
    for p in range(NU):
        nl = p // DUP
        t = ef[:, AXIS_LEN + nl * L2:AXIS_LEN + (nl + 1) * L2]
        rcols.append(_dot(t * outer[p * B:(p + 1) * B, :], o49))
    r = jnp.concatenate(rcols, axis=1)

    h = _leaky(pre[:, 64:128] + _dot(r, cwr[...]) + cb0[...])
    h = _leaky(_mm(h, c1[...], cb1[...]))
    h = _leaky(_mm(h, c2[...], cb2[...]))
    out = _mm(h, c3[...], cb3[...])
    k = jax.nn.sigmoid(out[:, 0:1])
    efc1_ref[...] = jnp.concatenate(
        [out[:, 1:1 + H1] * k, jnp.ones((B, 1), jnp.float32),
         jnp.zeros((B, 128 - H1 - 1), jnp.float32)], axis=1)
    efc2_ref[...] = out[:, 1 + H1:1 + H1 + H2] * k


def _node_body(nf_ref, netP_ref, cellP_ref, nfc2_ref, ats_ref,
               r0, rb0, r1, rb1, r2, rb2, r3, rb3, out_ref):
    nf = nf_ref[...]
    netp = netP_ref[0] + netP_ref[1]
    cellp = cellP_ref[0] + cellP_ref[1]
    nfc1 = cellp[:, 0:H1]
    w0 = r0[...]
    h = _leaky(_dot(nf, w0[0:IN_NF, :])
               + _dot(nfc1, w0[IN_NF:IN_NF + H1, :])
               + _dot(nfc2_ref[...], w0[IN_NF + H1:, :]) + rb0[...])
    h = _leaky(_mm(h, r1[...], rb1[...]))
    h = _leaky(_mm(h, r2[...], rb2[...]))
    red = _mm(h, r3[...], rb3[...])

    i = pl.program_id(0)
    row = jax.lax.broadcasted_iota(jnp.int32, (nf.shape[0], 1), 0) + i * nf.shape[0]
    base = jnp.where(row < N_PI, ats_ref[...], 0.0)
    v = jnp.where(netp[:, OUT_NF:OUT_NF + 1] > 0, netp[:, 0:OUT_NF], base)
    out_ref[...] = jnp.where(cellp[:, H1:H1 + 1] > 0, red, v)


def kernel(nf, n_atslew, cell_ef, netprop, lut_query, lut_att, cellarc,
           cellreduce, net_src, net_dst, cell_src, cell_dst):
    f32 = jnp.float32

    gp = N // B_NODE
    netT, srcT, dstT = pl.pallas_call(
        _proj_body,
        grid=(gp,),
        in_specs=[pl.BlockSpec((B_NODE, IN_NF), lambda i: (i, 0)),
                  pl.BlockSpec((B_NODE, OUT_NF), lambda i: (i, 0)),
                  _full_spec((264, 64)), _full_spec((264, 64)),
                  _full_spec((264, 64))],
        out_specs=[pl.BlockSpec((B_NODE, 128), lambda i: (i, 0)),
                   pl.BlockSpec((B_NODE, 128), lambda i: (i, 0)),
                   pl.BlockSpec((B_NODE, 128), lambda i: (i, 0))],
        out_shape=[jax.ShapeDtypeStruct((N, 128), f32),
                   jax.ShapeDtypeStruct((N, 128), f32),
                   jax.ShapeDtypeStruct((N, 128), f32)],
    )(nf, n_atslew, netprop[0][0], lut_query[0][0], cellarc[0][0][:264, :])

    netA, netB, cellA, cellB = _sc_gather(
        netT, srcT, dstT, net_src, net_dst, cell_src, cell_dst)

    gn = E_NET // B_NET
    np_b0 = [netprop[0][1].reshape(1, -1)]
    efn = pl.pallas_call(
        _net_body,
        grid=(gn,),
        in_specs=[pl.BlockSpec((B_NET, 128), lambda i: (i, 0)),
                  pl.BlockSpec((B_NET, 128), lambda i: (i, 0)),
                  _full_spec((1, 64))] + _wspecs(netprop[1:]),
        out_specs=pl.BlockSpec((B_NET, 128), lambda i: (i, 0)),
        out_shape=jax.ShapeDtypeStruct((E_NET, 128), f32),
    )(netA, netB, *np_b0, *_wargs(netprop[1:]))

    gc = E_CELL // B_CELL
    a0aug = jnp.concatenate([lut_att[0][0], lut_att[0][1][None, :]], axis=0)
    efc1, efc2 = pl.pallas_call(
        _cell_body,
        grid=(gc,),
        in_specs=[pl.BlockSpec((B_CELL, 128), lambda i: (i, 0)),
                  pl.BlockSpec((B_CELL, 128), lambda i: (i, 0)),
                  pl.BlockSpec((B_CELL, cell_ef.shape[1]), lambda i: (i, 0)),
                  _full_spec((L, L2)), _full_spec((L, L2)),
                  _full_spec((L2, 1)),
                  _full_spec((1, 64))] + _wspecs(lut_query[1:])
        + [_full_spec((2 + AXIS_W + 1, 64))] + _wspecs(lut_att[1:])
        + [_full_spec((NU, 64)), _full_spec((1, 64))] + _wspecs(cellarc[1:]),
        out_specs=[pl.BlockSpec((B_CELL, 128), lambda i: (i, 0)),
                   pl.BlockSpec((B_CELL, H2), lambda i: (i, 0))],
        out_shape=[jax.ShapeDtypeStruct((E_CELL, 128), f32),
                   jax.ShapeDtypeStruct((E_CELL, H2), f32)],
        scratch_shapes=[pltpu.VMEM((NU * B_CELL, 2 + AXIS_W + 1), f32)],
    )(cellA, cellB, cell_ef, jnp.asarray(_E1), jnp.asarray(_E2),
      jnp.ones((L2, 1), f32),
      lut_query[0][1].reshape(1, -1),
      *_wargs(lut_query[1:]), a0aug, *_wargs(lut_att[1:]),
      cellarc[0][0][264:280, :], cellarc[0][1].reshape(1, -1),
      *_wargs(cellarc[1:]))

    netP = _sc_scatter(efn, net_dst)
    cellP = _sc_scatter(efc1, cell_dst)

    nfc2 = jax.ops.segment_max(efc2, cell_dst, num_segments=N)
    nfc2 = jnp.where(jnp.isfinite(nfc2), nfc2, 0.0)

    gd = N // B_NODE
    out = pl.pallas_call(
        _node_body,
        grid=(gd,),
        in_specs=[pl.BlockSpec((B_NODE, IN_NF), lambda i: (i, 0)),
                  pl.BlockSpec((2, B_NODE, 128), lambda i: (0, i, 0)),
                  pl.BlockSpec((2, B_NODE, 128), lambda i: (0, i, 0)),
                  pl.BlockSpec((B_NODE, H2), lambda i: (i, 0)),
                  pl.BlockSpec((B_NODE, OUT_NF), lambda i: (i, 0))]
        + _wspecs(cellreduce),
        out_specs=pl.BlockSpec((B_NODE, OUT_NF), lambda i: (i, 0)),
        out_shape=jax.ShapeDtypeStruct((N, OUT_NF), f32),
    )(nf, netP, cellP, nfc2, n_atslew, *_wargs(cellreduce))
    return out


# speedup vs baseline: 1.0098x; 1.0098x over previous
import functools
import numpy as np
import jax
import jax.numpy as jnp
from jax import lax
from jax.experimental import pallas as pl
from jax.experimental.pallas import tpu as pltpu
from jax.experimental.pallas import tpu_sc as plsc

N = 10000
E_NET = 80000
E_CELL = 80000
IN_NF = 128
OUT_NF = 8
NL = 4
L = 7
DUP = 4
H1 = 32
H2 = 32
N_PI = 1000
NU = NL * DUP
AXIS_W = 1 + 2 * L
L2 = L * L
AXIS_LEN = NL * AXIS_W

B_NET = 2000
B_CELL = 640
B_NODE = 2000

_E1 = np.zeros((L, L2), np.float32)
_E2 = np.zeros((L, L2), np.float32)
for _i in range(L):
    for _j in range(L):
        _E1[_i, _i * L + _j] = 1.0
        _E2[_j, _i * L + _j] = 1.0


def _leaky(x):
    return jnp.maximum(x, 0.2 * x)


def _dot(x, w):
    return jax.lax.dot_general(x, w, (((1,), (0,)), ((), ())),
                               preferred_element_type=jnp.float32)


def _mm(x, w, b):
    return _dot(x, w) + b


def _full_spec(shape):
    return pl.BlockSpec(shape, lambda i: tuple(0 for _ in shape))


def _wspecs(params):
    specs = []
    for (w, b) in params:
        specs.append(_full_spec(w.shape))
        specs.append(_full_spec((1, b.shape[0])))
    return specs


def _wargs(params):
    args = []
    for (w, b) in params:
        args.append(w)
        args.append(b.reshape(1, -1))
    return args


GK = 128
NCH = E_NET // GK
NWORK = 32
MAXJ = (NCH + NWORK - 1) // NWORK


def _sc_gather_body(netT_h, srcT_h, dstT_h,
                    ns_h, nd_h, cs_h, cd_h,
                    oNS_h, oND_h, oCS_h, oCD_h,
                    idx_v, b64_a, b64_b, b128_a, b128_b,
                    sem_a, sem_b):
    wid = lax.axis_index("s") * 2 + lax.axis_index("c")
    streams = [
        (netT_h, ns_h, oNS_h, (b64_a, b64_b)),
        (netT_h, nd_h, oND_h, (b64_a, b64_b)),
        (srcT_h, cs_h, oCS_h, (b128_a, b128_b)),
        (dstT_h, cd_h, oCD_h, (b128_a, b128_b)),
    ]
    semb = (sem_a, sem_b)
    for tab, idx2, out, bufs in streams:
        for j in range(MAXJ):
            ch = j * NWORK + wid

            @pl.when(ch < NCH)
            def _(ch=ch, tab=tab, idx2=idx2, out=out,
                  buf=bufs[j % 2], sem=semb[j % 2]):
                pltpu.sync_copy(idx2.at[ch], idx_v)
                pltpu.async_copy(tab.at[idx_v], buf, sem).wait()
                pltpu.sync_copy(buf, out.at[pl.ds(ch * GK, GK)])


def _sc_gather(netT, srcT, dstT, net_src, net_dst, cell_src, cell_dst):
    f32 = jnp.float32
    i32 = jnp.int32
    idx2 = [a.reshape(NCH, GK).astype(i32)
            for a in (net_src, net_dst, cell_src, cell_dst)]
    mesh = plsc.VectorSubcoreMesh(core_axis_name="c", subcore_axis_name="s")
    k = functools.partial(
        pl.kernel, mesh=mesh,
        out_type=[jax.ShapeDtypeStruct((E_NET, 128), f32),
                  jax.ShapeDtypeStruct((E_NET, 128), f32),
                  jax.ShapeDtypeStruct((E_CELL, 128), f32),
                  jax.ShapeDtypeStruct((E_CELL, 128), f32)],
        scratch_types=[pltpu.VMEM((GK,), i32),
                       pltpu.VMEM((GK, 128), f32),
                       pltpu.VMEM((GK, 128), f32),
                       pltpu.VMEM((GK, 128), f32),
                       pltpu.VMEM((GK, 128), f32),
                       pltpu.SemaphoreType.DMA,
                       pltpu.SemaphoreType.DMA],
    )(_sc_gather_body)
    return k(netT, srcT, dstT, *idx2)


WN = 16
WC = 48
ZCH = 10


def _sc_scat_body(vals_h, dst_h, z_h, out_h, idx_v, vb, acc):
    cid = lax.axis_index("c")
    sid = lax.axis_index("s")
    wid = sid * 2 + cid

    @pl.when(sid < ZCH)
    def _():
        pltpu.sync_copy(z_h.at[pl.ds(sid * 1000, 1000)],
                        acc.at[pl.ds(sid * 1000, 1000)])

    plsc.subcore_barrier()

    for j in range(MAXJ):
        ch = j * NWORK + wid

        @pl.when(ch < NCH)
        def _(ch=ch):
            pltpu.sync_copy(vals_h.at[pl.ds(ch * GK, GK)], vb)
            pltpu.sync_copy(dst_h.at[ch], idx_v)
            pltpu.sync_copy(vb, acc.at[idx_v], add=True)

    plsc.subcore_barrier()

    @pl.when(sid < ZCH)
    def _():
        pltpu.sync_copy(acc.at[pl.ds(sid * 1000, 1000)],
                        out_h.at[cid, pl.ds(sid * 1000, 1000)])


def _sc_scatter(vals128, dst):
    f32 = jnp.float32
    dst2 = dst.reshape(NCH, GK).astype(jnp.int32)
    mesh = plsc.VectorSubcoreMesh(core_axis_name="c", subcore_axis_name="s")
    k = functools.partial(
        pl.kernel, mesh=mesh,
        out_type=jax.ShapeDtypeStruct((2, N, 128), f32),
        scratch_types=[pltpu.VMEM((GK,), jnp.int32),
                       pltpu.VMEM((GK, 128), f32),
                       pltpu.VMEM_SHARED((N, 128), f32)],
    )(_sc_scat_body)
    return k(vals128, dst2, jnp.zeros((N, 128), f32))


def _proj_body(nf_ref, ats_ref, nw_ref, qw_ref, cw_ref,
               netT_ref, srcT_ref, dstT_ref):
    nf = nf_ref[...]
    ats = ats_ref[...]
    nw = nw_ref[...]
    qw = qw_ref[...]
    cw = cw_ref[...]
    netS = _dot(ats, nw[0:OUT_NF, :]) + _dot(nf, nw[OUT_NF:OUT_NF + IN_NF, :])
    netD = _dot(nf, nw[OUT_NF + IN_NF:, :])
    netT_ref[...] = jnp.concatenate([netS, netD], axis=1)
    qS = _dot(ats, qw[0:OUT_NF, :]) + _dot(nf, qw[OUT_NF:OUT_NF + IN_NF, :])
    qD = _dot(nf, qw[OUT_NF + IN_NF:, :])
    cS = _dot(ats, cw[0:OUT_NF, :]) + _dot(nf, cw[OUT_NF:OUT_NF + IN_NF, :])
    cD = _dot(nf, cw[OUT_NF + IN_NF:, :])
    srcT_ref[...] = jnp.concatenate([qS, cS], axis=1)
    dstT_ref[...] = jnp.concatenate([qD, cD], axis=1)


def _net_body(preA_ref, preB_ref, b0, w1, b1, w2, b2, w3, b3, w4, b4, out_ref):
    h = _leaky(preA_ref[:, 0:64] + preB_ref[:, 64:128] + b0[...])
    h = _leaky(_mm(h, w1[...], b1[...]))
    h = _leaky(_mm(h, w2[...], b2[...]))
    h = _leaky(_mm(h, w3[...], b3[...]))
    efn = _mm(h, w4[...], b4[...])
    B = efn.shape[0]
    out_ref[...] = jnp.concatenate(
        [efn, jnp.ones((B, 1), jnp.float32),
         jnp.zeros((B, 128 - OUT_NF - 1), jnp.float32)], axis=1)


def _cell_body(preA_ref, preB_ref, ef_ref, e1_ref, e2_ref, o49_ref,
               qb0, q1, qb1, q2, qb2, q3, qb3,
               a0, a1, ab1, a2, ab2, a3, ab3,
               cwr, cb0, c1, cb1, c2, cb2, c3, cb3,
               efc1_ref, efc2_ref, x1_scr):
    pre = preA_ref[...] + preB_ref[...]
    ef = ef_ref[...]

    h = _leaky(pre[:, 0:64] + qb0[...])
    h = _leaky(_mm(h, q1[...], qb1[...]))
    h = _leaky(_mm(h, q2[...], qb2[...]))
    q = _mm(h, q3[...], qb3[...])

    B = q.shape[0]

    @pl.when(pl.program_id(0) == 0)
    def _():
        x1_scr[:, 2 + AXIS_W:] = jnp.ones((NU * B, 1), jnp.float32)

    for p in range(NU):
        nl = p // DUP
        x1_scr[p * B:(p + 1) * B, 0:2] = q[:, 2 * p:2 * p + 2]
        x1_scr[p * B:(p + 1) * B, 2:2 + AXIS_W] = (
            ef[:, nl * AXIS_W:(nl + 1) * AXIS_W])

    xa = x1_scr[...]
    h = _leaky(_dot(xa, a0[...]))
    h = _leaky(_mm(h, a1[...], ab1[...]))
    h = _leaky(_mm(h, a2[...], ab2[...]))
    a4 = _mm(h, a3[...], ab3[...])

    ax = a4[:, 0:L]
    ay = a4[:, L:2 * L]
    outer = _dot(ax, e1_ref[...]) * _dot(ay, e2_ref[...])
    tabs = jnp.concatenate(
        [ef[:, AXIS_LEN + (p // DUP) * L2:AXIS_LEN + (p // DUP + 1) * L2]
         for p in range(NU)], axis=0)
    rcol = _dot(tabs * outer, o49_ref[...])
    r = jnp.concatenate(
        [rcol[p * B:(p + 1) * B, :] for p in range(NU)], axis=1)

    h = _leaky(pre[:, 64:128] + _dot(r, cwr[...]) + cb0[...])
    h = _leaky(_mm(h, c1[...], cb1[...]))
    h = _leaky(_mm(h, c2[...], cb2[...]))
    out = _mm(h, c3[...], cb3[...])
    k = jax.nn.sigmoid(out[:, 0:1])
    efc1_ref[...] = jnp.concatenate(
        [out[:, 1:1 + H1] * k, jnp.ones((B, 1), jnp.float32),
         jnp.zeros((B, 128 - H1 - 1), jnp.float32)], axis=1)
    efc2_ref[...] = out[:, 1 + H1:1 + H1 + H2] * k


def _node_body(nf_ref, netP_ref, cellP_ref, nfc2_ref, ats_ref,
               r0, rb0, r1, rb1, r2, rb2, r3, rb3, out_ref):
    nf = nf_ref[...]
    netp = netP_ref[0] + netP_ref[1]
    cellp = cellP_ref[0] + cellP_ref[1]
    nfc1 = cellp[:, 0:H1]
    w0 = r0[...]
    h = _leaky(_dot(nf, w0[0:IN_NF, :])
               + _dot(nfc1, w0[IN_NF:IN_NF + H1, :])
               + _dot(nfc2_ref[...], w0[IN_NF + H1:, :]) + rb0[...])
    h = _leaky(_mm(h, r1[...], rb1[...]))
    h = _leaky(_mm(h, r2[...], rb2[...]))
    red = _mm(h, r3[...], rb3[...])

    i = pl.program_id(0)
    row = jax.lax.broadcasted_iota(jnp.int32, (nf.shape[0], 1), 0) + i * nf.shape[0]
    base = jnp.where(row < N_PI, ats_ref[...], 0.0)
    v = jnp.where(netp[:, OUT_NF:OUT_NF + 1] > 0, netp[:, 0:OUT_NF], base)
    out_ref[...] = jnp.where(cellp[:, H1:H1 + 1] > 0, red, v)


def kernel(nf, n_atslew, cell_ef, netprop, lut_query, lut_att, cellarc,
           cellreduce, net_src, net_dst, cell_src, cell_dst):
    f32 = jnp.float32

    gp = N // B_NODE
    netT, srcT, dstT = pl.pallas_call(
        _proj_body,
        grid=(gp,),
        in_specs=[pl.BlockSpec((B_NODE, IN_NF), lambda i: (i, 0)),
                  pl.BlockSpec((B_NODE, OUT_NF), lambda i: (i, 0)),
                  _full_spec((264, 64)), _full_spec((264, 64)),
                  _full_spec((264, 64))],
        out_specs=[pl.BlockSpec((B_NODE, 128), lambda i: (i, 0)),
                   pl.BlockSpec((B_NODE, 128), lambda i: (i, 0)),
                   pl.BlockSpec((B_NODE, 128), lambda i: (i, 0))],
        out_shape=[jax.ShapeDtypeStruct((N, 128), f32),
                   jax.ShapeDtypeStruct((N, 128), f32),
                   jax.ShapeDtypeStruct((N, 128), f32)],
    )(nf, n_atslew, netprop[0][0], lut_query[0][0], cellarc[0][0][:264, :])

    netA, netB, cellA, cellB = _sc_gather(
        netT, srcT, dstT, net_src, net_dst, cell_src, cell_dst)

    gn = E_NET // B_NET
    np_b0 = [netprop[0][1].reshape(1, -1)]
    efn = pl.pallas_call(
        _net_body,
        grid=(gn,),
        in_specs=[pl.BlockSpec((B_NET, 128), lambda i: (i, 0)),
                  pl.BlockSpec((B_NET, 128), lambda i: (i, 0)),
                  _full_spec((1, 64))] + _wspecs(netprop[1:]),
        out_specs=pl.BlockSpec((B_NET, 128), lambda i: (i, 0)),
        out_shape=jax.ShapeDtypeStruct((E_NET, 128), f32),
    )(netA, netB, *np_b0, *_wargs(netprop[1:]))

    gc = E_CELL // B_CELL
    a0aug = jnp.concatenate([lut_att[0][0], lut_att[0][1][None, :]], axis=0)
    efc1, efc2 = pl.pallas_call(
        _cell_body,
        grid=(gc,),
        in_specs=[pl.BlockSpec((B_CELL, 128), lambda i: (i, 0)),
                  pl.BlockSpec((B_CELL, 128), lambda i: (i, 0)),
                  pl.BlockSpec((B_CELL, cell_ef.shape[1]), lambda i: (i, 0)),
                  _full_spec((L, L2)), _full_spec((L, L2)),
                  _full_spec((L2, 1)),
                  _full_spec((1, 64))] + _wspecs(lut_query[1:])
        + [_full_spec((2 + AXIS_W + 1, 64))] + _wspecs(lut_att[1:])
        + [_full_spec((NU, 64)), _full_spec((1, 64))] + _wspecs(cellarc[1:]),
        out_specs=[pl.BlockSpec((B_CELL, 128), lambda i: (i, 0)),
                   pl.BlockSpec((B_CELL, H2), lambda i: (i, 0))],
        out_shape=[jax.ShapeDtypeStruct((E_CELL, 128), f32),
                   jax.ShapeDtypeStruct((E_CELL, H2), f32)],
        scratch_shapes=[pltpu.VMEM((NU * B_CELL, 2 + AXIS_W + 1), f32)],
    )(cellA, cellB, cell_ef, jnp.asarray(_E1), jnp.asarray(_E2),
      jnp.ones((L2, 1), f32),
      lut_query[0][1].reshape(1, -1),
      *_wargs(lut_query[1:]), a0aug, *_wargs(lut_att[1:]),
      cellarc[0][0][264:280, :], cellarc[0][1].reshape(1, -1),
      *_wargs(cellarc[1:]))

    netP = _sc_scatter(efn, net_dst)
    cellP = _sc_scatter(efc1, cell_dst)

    nfc2 = jax.ops.segment_max(efc2, cell_dst, num_segments=N)
    nfc2 = jnp.where(jnp.isfinite(nfc2), nfc2, 0.0)

    gd = N // B_NODE
    out = pl.pallas_call(
        _node_body,
        grid=(gd,),
        in_specs=[pl.BlockSpec((B_NODE, IN_NF), lambda i: (i, 0)),
                  pl.BlockSpec((2, B_NODE, 128), lambda i: (0, i, 0)),
                  pl.BlockSpec((2, B_NODE, 128), lambda i: (0, i, 0)),
                  pl.BlockSpec((B_NODE, H2), lambda i: (i, 0)),
                  pl.BlockSpec((B_NODE, OUT_NF), lambda i: (i, 0))]
        + _wspecs(cellreduce),
        out_specs=pl.BlockSpec((B_NODE, OUT_NF), lambda i: (i, 0)),
        out_shape=jax.ShapeDtypeStruct((N, OUT_NF), f32),
    )(nf, netP, cellP, nfc2, n_atslew, *_wargs(cellreduce))
    return out


# speedup vs baseline: 1.0722x; 1.0618x over previous
import functools
import numpy as np
import jax
import jax.numpy as jnp
from jax import lax
from jax.experimental import pallas as pl
from jax.experimental.pallas import tpu as pltpu
from jax.experimental.pallas import tpu_sc as plsc

N = 10000
E_NET = 80000
E_CELL = 80000
IN_NF = 128
OUT_NF = 8
NL = 4
L = 7
DUP = 4
H1 = 32
H2 = 32
N_PI = 1000
NU = NL * DUP
AXIS_W = 1 + 2 * L
L2 = L * L
AXIS_LEN = NL * AXIS_W

B_NET = 2000
B_CELL = 640
B_NODE = 2000

_E1 = np.zeros((L, L2), np.float32)
_E2 = np.zeros((L, L2), np.float32)
for _i in range(L):
    for _j in range(L):
        _E1[_i, _i * L + _j] = 1.0
        _E2[_j, _i * L + _j] = 1.0


def _leaky(x):
    return jnp.maximum(x, 0.2 * x)


def _dot(x, w):
    return jax.lax.dot_general(x, w, (((1,), (0,)), ((), ())),
                               preferred_element_type=jnp.float32)


def _mm(x, w, b):
    return _dot(x, w) + b


def _full_spec(shape):
    return pl.BlockSpec(shape, lambda i: tuple(0 for _ in shape))


def _wspecs(params):
    specs = []
    for (w, b) in params:
        specs.append(_full_spec(w.shape))
        specs.append(_full_spec((1, b.shape[0])))
    return specs


def _wargs(params):
    args = []
    for (w, b) in params:
        args.append(w)
        args.append(b.reshape(1, -1))
    return args


GK = 128
NCH = E_NET // GK
NWORK = 32
MAXJ = (NCH + NWORK - 1) // NWORK


def _sc_gather_body(netT_h, srcT_h, dstT_h,
                    ns_h, nd_h, cs_h, cd_h,
                    oNS_h, oND_h, oCS_h, oCD_h,
                    idx_v, b64_a, b64_b, b128_a, b128_b,
                    sem_a, sem_b):
    wid = lax.axis_index("s") * 2 + lax.axis_index("c")
    streams = [
        (netT_h, ns_h, oNS_h, (b64_a, b64_b)),
        (netT_h, nd_h, oND_h, (b64_a, b64_b)),
        (srcT_h, cs_h, oCS_h, (b128_a, b128_b)),
        (dstT_h, cd_h, oCD_h, (b128_a, b128_b)),
    ]
    semb = (sem_a, sem_b)
    for tab, idx2, out, bufs in streams:
        for j in range(MAXJ):
            ch = j * NWORK + wid

            @pl.when(ch < NCH)
            def _(ch=ch, tab=tab, idx2=idx2, out=out,
                  buf=bufs[j % 2], sem=semb[j % 2]):
                pltpu.sync_copy(idx2.at[ch], idx_v)
                pltpu.async_copy(tab.at[idx_v], buf, sem).wait()
                pltpu.sync_copy(buf, out.at[pl.ds(ch * GK, GK)])


def _sc_gather(netT, srcT, dstT, net_src, net_dst, cell_src, cell_dst):
    f32 = jnp.float32
    i32 = jnp.int32
    idx2 = [a.reshape(NCH, GK).astype(i32)
            for a in (net_src, net_dst, cell_src, cell_dst)]
    mesh = plsc.VectorSubcoreMesh(core_axis_name="c", subcore_axis_name="s")
    k = functools.partial(
        pl.kernel, mesh=mesh,
        out_type=[jax.ShapeDtypeStruct((E_NET, 128), f32),
                  jax.ShapeDtypeStruct((E_NET, 128), f32),
                  jax.ShapeDtypeStruct((E_CELL, 128), f32),
                  jax.ShapeDtypeStruct((E_CELL, 128), f32)],
        scratch_types=[pltpu.VMEM((GK,), i32),
                       pltpu.VMEM((GK, 128), f32),
                       pltpu.VMEM((GK, 128), f32),
                       pltpu.VMEM((GK, 128), f32),
                       pltpu.VMEM((GK, 128), f32),
                       pltpu.SemaphoreType.DMA,
                       pltpu.SemaphoreType.DMA],
    )(_sc_gather_body)
    return k(netT, srcT, dstT, *idx2)


WN = 16
WC = 48
ZCH = 10


def _sc_scat_body(vals_h, dst_h, z_h, out_h, idx_v, vb, acc):
    cid = lax.axis_index("c")
    sid = lax.axis_index("s")
    wid = sid * 2 + cid

    @pl.when(sid < ZCH)
    def _():
        pltpu.sync_copy(z_h.at[pl.ds(sid * 1000, 1000)],
                        acc.at[pl.ds(sid * 1000, 1000)])

    plsc.subcore_barrier()

    for j in range(MAXJ):
        ch = j * NWORK + wid

        @pl.when(ch < NCH)
        def _(ch=ch):
            pltpu.sync_copy(vals_h.at[pl.ds(ch * GK, GK)], vb)
            pltpu.sync_copy(dst_h.at[ch], idx_v)
            pltpu.sync_copy(vb, acc.at[idx_v], add=True)

    plsc.subcore_barrier()

    @pl.when(sid < ZCH)
    def _():
        pltpu.sync_copy(acc.at[pl.ds(sid * 1000, 1000)],
                        out_h.at[cid, pl.ds(sid * 1000, 1000)])


def _sc_scatter(vals128, dst):
    f32 = jnp.float32
    dst2 = dst.reshape(NCH, GK).astype(jnp.int32)
    mesh = plsc.VectorSubcoreMesh(core_axis_name="c", subcore_axis_name="s")
    k = functools.partial(
        pl.kernel, mesh=mesh,
        out_type=jax.ShapeDtypeStruct((2, N, 128), f32),
        scratch_types=[pltpu.VMEM((GK,), jnp.int32),
                       pltpu.VMEM((GK, 128), f32),
                       pltpu.VMEM_SHARED((N, 128), f32)],
    )(_sc_scat_body)
    return k(vals128, dst2, jnp.zeros((N, 128), f32))


def _proj_body(nf_ref, ats_ref, nw_ref, qw_ref, cw_ref,
               netT_ref, srcT_ref, dstT_ref):
    nf = nf_ref[...]
    ats = ats_ref[...]
    nw = nw_ref[...]
    qw = qw_ref[...]
    cw = cw_ref[...]
    netS = _dot(ats, nw[0:OUT_NF, :]) + _dot(nf, nw[OUT_NF:OUT_NF + IN_NF, :])
    netD = _dot(nf, nw[OUT_NF + IN_NF:, :])
    netT_ref[...] = jnp.concatenate([netS, netD], axis=1)
    qS = _dot(ats, qw[0:OUT_NF, :]) + _dot(nf, qw[OUT_NF:OUT_NF + IN_NF, :])
    qD = _dot(nf, qw[OUT_NF + IN_NF:, :])
    cS = _dot(ats, cw[0:OUT_NF, :]) + _dot(nf, cw[OUT_NF:OUT_NF + IN_NF, :])
    cD = _dot(nf, cw[OUT_NF + IN_NF:, :])
    srcT_ref[...] = jnp.concatenate([qS, cS], axis=1)
    dstT_ref[...] = jnp.concatenate([qD, cD], axis=1)


def _net_body(preA_ref, preB_ref, b0, w1, b1, w2, b2, w3, b3, w4, b4, out_ref):
    h = _leaky(preA_ref[:, 0:64] + preB_ref[:, 64:128] + b0[...])
    h = _leaky(_mm(h, w1[...], b1[...]))
    h = _leaky(_mm(h, w2[...], b2[...]))
    h = _leaky(_mm(h, w3[...], b3[...]))
    efn = _mm(h, w4[...], b4[...])
    B = efn.shape[0]
    out_ref[...] = jnp.concatenate(
        [efn, jnp.ones((B, 1), jnp.float32),
         jnp.zeros((B, 128 - OUT_NF - 1), jnp.float32)], axis=1)


def _cell_body(preA_ref, preB_ref, ef_ref, e1_ref, e2_ref,
               qb0, q1, qb1, q2, qb2, q3, qb3,
               a0, ab0, a1, ab1, a2, ab2, a3, ab3,
               cwr, cb0, c1, cb1, c2, cb2, c3, cb3,
               efc1_ref, efc2_ref, x1_scr):
    pre = preA_ref[...] + preB_ref[...]
    ef = ef_ref[...]

    h = _leaky(pre[:, 0:64] + qb0[...])
    h = _leaky(_mm(h, q1[...], qb1[...]))
    h = _leaky(_mm(h, q2[...], qb2[...]))
    q = _mm(h, q3[...], qb3[...])

    B = q.shape[0]
    for p in range(NU):
        nl = p // DUP
        x1_scr[p * B:(p + 1) * B, 0:2] = q[:, 2 * p:2 * p + 2]
        x1_scr[p * B:(p + 1) * B, 2:2 + AXIS_W] = (
            ef[:, nl * AXIS_W:(nl + 1) * AXIS_W])

    xa = x1_scr[...]
    h = _leaky(_mm(xa, a0[...], ab0[...]))
    h = _leaky(_mm(h, a1[...], ab1[...]))
    h = _leaky(_mm(h, a2[...], ab2[...]))
    a4 = _mm(h, a3[...], ab3[...])

    ax = a4[:, 0:L]
    ay = a4[:, L:2 * L]
    outer = _dot(ax, e1_ref[...]) * _dot(ay, e2_ref[...])
    tabs = jnp.concatenate(
        [ef[:, AXIS_LEN + (p // DUP) * L2:AXIS_LEN + (p // DUP + 1) * L2]
         for p in range(NU)], axis=0)
    rcol = jnp.sum(tabs * outer, axis=1, keepdims=True)
    r = jnp.concatenate(
        [rcol[p * B:(p + 1) * B, :] for p in range(NU)], axis=1)

    h = _leaky(pre[:, 64:128] + _dot(r, cwr[...]) + cb0[...])
    h = _leaky(_mm(h, c1[...], cb1[...]))
    h = _leaky(_mm(h, c2[...], cb2[...]))
    out = _mm(h, c3[...], cb3[...])
    k = jax.nn.sigmoid(out[:, 0:1])
    efc1_ref[...] = jnp.concatenate(
        [out[:, 1:1 + H1] * k, jnp.ones((B, 1), jnp.float32),
         jnp.zeros((B, 128 - H1 - 1), jnp.float32)], axis=1)
    efc2_ref[...] = out[:, 1 + H1:1 + H1 + H2] * k


def _node_body(nf_ref, netP_ref, cellP_ref, nfc2_ref, ats_ref,
               r0, rb0, r1, rb1, r2, rb2, r3, rb3, out_ref):
    nf = nf_ref[...]
    netp = netP_ref[0] + netP_ref[1]
    cellp = cellP_ref[0] + cellP_ref[1]
    nfc1 = cellp[:, 0:H1]
    w0 = r0[...]
    h = _leaky(_dot(nf, w0[0:IN_NF, :])
               + _dot(nfc1, w0[IN_NF:IN_NF + H1, :])
               + _dot(nfc2_ref[...], w0[IN_NF + H1:, :]) + rb0[...])
    h = _leaky(_mm(h, r1[...], rb1[...]))
    h = _leaky(_mm(h, r2[...], rb2[...]))
    red = _mm(h, r3[...], rb3[...])

    i = pl.program_id(0)
    row = jax.lax.broadcasted_iota(jnp.int32, (nf.shape[0], 1), 0) + i * nf.shape[0]
    base = jnp.where(row < N_PI, ats_ref[...], 0.0)
    v = jnp.where(netp[:, OUT_NF:OUT_NF + 1] > 0, netp[:, 0:OUT_NF], base)
    out_ref[...] = jnp.where(cellp[:, H1:H1 + 1] > 0, red, v)


def kernel(nf, n_atslew, cell_ef, netprop, lut_query, lut_att, cellarc,
           cellreduce, net_src, net_dst, cell_src, cell_dst):
    f32 = jnp.float32

    gp = N // B_NODE
    netT, srcT, dstT = pl.pallas_call(
        _proj_body,
        grid=(gp,),
        in_specs=[pl.BlockSpec((B_NODE, IN_NF), lambda i: (i, 0)),
                  pl.BlockSpec((B_NODE, OUT_NF), lambda i: (i, 0)),
                  _full_spec((264, 64)), _full_spec((264, 64)),
                  _full_spec((264, 64))],
        out_specs=[pl.BlockSpec((B_NODE, 128), lambda i: (i, 0)),
                   pl.BlockSpec((B_NODE, 128), lambda i: (i, 0)),
                   pl.BlockSpec((B_NODE, 128), lambda i: (i, 0))],
        out_shape=[jax.ShapeDtypeStruct((N, 128), f32),
                   jax.ShapeDtypeStruct((N, 128), f32),
                   jax.ShapeDtypeStruct((N, 128), f32)],
    )(nf, n_atslew, netprop[0][0], lut_query[0][0], cellarc[0][0][:264, :])

    netA, netB, cellA, cellB = _sc_gather(
        netT, srcT, dstT, net_src, net_dst, cell_src, cell_dst)

    gn = E_NET // B_NET
    np_b0 = [netprop[0][1].reshape(1, -1)]
    efn = pl.pallas_call(
        _net_body,
        grid=(gn,),
        in_specs=[pl.BlockSpec((B_NET, 128), lambda i: (i, 0)),
                  pl.BlockSpec((B_NET, 128), lambda i: (i, 0)),
                  _full_spec((1, 64))] + _wspecs(netprop[1:]),
        out_specs=pl.BlockSpec((B_NET, 128), lambda i: (i, 0)),
        out_shape=jax.ShapeDtypeStruct((E_NET, 128), f32),
    )(netA, netB, *np_b0, *_wargs(netprop[1:]))

    gc = E_CELL // B_CELL
    efc1, efc2 = pl.pallas_call(
        _cell_body,
        grid=(gc,),
        in_specs=[pl.BlockSpec((B_CELL, 128), lambda i: (i, 0)),
                  pl.BlockSpec((B_CELL, 128), lambda i: (i, 0)),
                  pl.BlockSpec((B_CELL, cell_ef.shape[1]), lambda i: (i, 0)),
                  _full_spec((L, L2)), _full_spec((L, L2)),
                  _full_spec((1, 64))] + _wspecs(lut_query[1:])
        + _wspecs(lut_att)
        + [_full_spec((NU, 64)), _full_spec((1, 64))] + _wspecs(cellarc[1:]),
        out_specs=[pl.BlockSpec((B_CELL, 128), lambda i: (i, 0)),
                   pl.BlockSpec((B_CELL, H2), lambda i: (i, 0))],
        out_shape=[jax.ShapeDtypeStruct((E_CELL, 128), f32),
                   jax.ShapeDtypeStruct((E_CELL, H2), f32)],
        scratch_shapes=[pltpu.VMEM((NU * B_CELL, 2 + AXIS_W), f32)],
    )(cellA, cellB, cell_ef, jnp.asarray(_E1), jnp.asarray(_E2),
      lut_query[0][1].reshape(1, -1),
      *_wargs(lut_query[1:]), *_wargs(lut_att),
      cellarc[0][0][264:280, :], cellarc[0][1].reshape(1, -1),
      *_wargs(cellarc[1:]))

    netP = _sc_scatter(efn, net_dst)
    cellP = _sc_scatter(efc1, cell_dst)

    nfc2 = jax.ops.segment_max(efc2, cell_dst, num_segments=N)
    nfc2 = jnp.where(jnp.isfinite(nfc2), nfc2, 0.0)

    gd = N // B_NODE
    out = pl.pallas_call(
        _node_body,
        grid=(gd,),
        in_specs=[pl.BlockSpec((B_NODE, IN_NF), lambda i: (i, 0)),
                  pl.BlockSpec((2, B_NODE, 128), lambda i: (0, i, 0)),
                  pl.BlockSpec((2, B_NODE, 128), lambda i: (0, i, 0)),
                  pl.BlockSpec((B_NODE, H2), lambda i: (i, 0)),
                  pl.BlockSpec((B_NODE, OUT_NF), lambda i: (i, 0))]
        + _wspecs(cellreduce),
        out_specs=pl.BlockSpec((B_NODE, OUT_NF), lambda i: (i, 0)),
        out_shape=jax.ShapeDtypeStruct((N, OUT_NF), f32),
    )(nf, netP, cellP, nfc2, n_atslew, *_wargs(cellreduce))
    return out
